# Initial kernel scaffold; baseline (speedup 1.0000x reference)
import jax
import jax.numpy as jnp
from jax.experimental import pallas as pl


def kernel(x, edge_index, W_l, b_l, W_r):
    raise NotImplementedError("write your pallas kernel here")


# speedup vs baseline: 5.6026x; 5.6026x over previous
import functools

import jax
import jax.numpy as jnp
from jax import lax
from jax.experimental import pallas as pl
from jax.experimental.pallas import tpu as pltpu
from jax.experimental.pallas import tpu_sc as plsc

NC = 2
NS = 16
NW = NC * NS
CHUNK = 80


def _sc_aggregate(xaug, src, dst, zinit, n_pad, da):
    e = src.shape[0]
    epw = e // NW
    nchunk = epw // CHUNK
    rows_per_tile = n_pad // NS

    mesh = plsc.VectorSubcoreMesh(core_axis_name="c", subcore_axis_name="s")

    @functools.partial(
        pl.kernel,
        out_type=jax.ShapeDtypeStruct((NC, n_pad, da), jnp.float32),
        mesh=mesh,
        scratch_types=[
            pltpu.VMEM((CHUNK,), jnp.int32),
            pltpu.VMEM((CHUNK,), jnp.int32),
            pltpu.VMEM((CHUNK, da), jnp.float32),
            pltpu.VMEM_SHARED((n_pad, da), jnp.float32),
            pltpu.SemaphoreType.DMA,
        ],
        compiler_params=pltpu.CompilerParams(use_tc_tiling_on_sc=False),
    )
    def agg(xaug_hbm, src_hbm, dst_hbm, zero_hbm, out_hbm,
            sidx_v, didx_v, rows_v, acc_sh, sem):
        cid = lax.axis_index("c")
        sid = lax.axis_index("s")
        wid = sid * NC + cid
        base = wid * epw
        t0 = sid * rows_per_tile

        pltpu.sync_copy(zero_hbm.at[pl.ds(t0, rows_per_tile)],
                        acc_sh.at[pl.ds(t0, rows_per_tile)])
        plsc.subcore_barrier()

        def body(j, carry):
            off = base + j * CHUNK
            pltpu.sync_copy(src_hbm.at[pl.ds(off, CHUNK)], sidx_v)
            pltpu.sync_copy(dst_hbm.at[pl.ds(off, CHUNK)], didx_v)
            pltpu.async_copy(xaug_hbm.at[sidx_v], rows_v, sem).wait()
            pltpu.sync_copy(rows_v, acc_sh.at[didx_v], add=True)
            return carry

        lax.fori_loop(0, nchunk, body, 0)
        plsc.subcore_barrier()

        pltpu.sync_copy(acc_sh.at[pl.ds(t0, rows_per_tile)],
                        out_hbm.at[cid, pl.ds(t0, rows_per_tile)])

    return agg(xaug, src, dst, zinit)


def _tc_finish_body(p_ref, x_ref, wl_ref, b_ref, wr_ref, o_ref, *, d):
    p = p_ref[...]
    summed = p[0, :, :d] + p[1, :, :d]
    cnt = p[0, :, d] + p[1, :, d]
    mean = summed / jnp.maximum(cnt, 1.0)[:, None]
    o_ref[...] = (
        jnp.dot(mean, wl_ref[...], preferred_element_type=jnp.float32)
        + b_ref[...]
        + jnp.dot(x_ref[...], wr_ref[...], preferred_element_type=jnp.float32)
    )


def kernel(x, edge_index, W_l, b_l, W_r):
    n, d = x.shape
    h = W_l.shape[1]
    e = edge_index.shape[1]
    da = ((d + 1 + 15) // 16) * 16
    n_pad = ((n + 8 * NW - 1) // (8 * NW)) * (8 * NW)
    assert e % (NW * CHUNK) == 0

    xaug = jnp.pad(
        jnp.concatenate([x, jnp.ones((n, 1), x.dtype)], axis=1),
        ((0, n_pad - n), (0, da - d - 1)),
    )
    src = edge_index[0]
    dst = edge_index[1]
    zinit = jnp.zeros((n_pad, da), jnp.float32)

    partial = _sc_aggregate(xaug, src, dst, zinit, n_pad, da)

    blk = 1000
    grid = (n // blk,)
    out = pl.pallas_call(
        functools.partial(_tc_finish_body, d=d),
        grid=grid,
        in_specs=[
            pl.BlockSpec((NC, blk, da), lambda i: (0, i, 0)),
            pl.BlockSpec((blk, d), lambda i: (i, 0)),
            pl.BlockSpec((d, h), lambda i: (0, 0)),
            pl.BlockSpec((1, h), lambda i: (0, 0)),
            pl.BlockSpec((d, h), lambda i: (0, 0)),
        ],
        out_specs=pl.BlockSpec((blk, h), lambda i: (i, 0)),
        out_shape=jax.ShapeDtypeStruct((n, h), jnp.float32),
    )(partial, x, W_l, b_l.reshape(1, h), W_r)
    return out
